# baseline (speedup 1.0000x reference)
import dataclasses
import functools

import jax
import jax.numpy as jnp
from jax import lax
from jax.experimental import pallas as pl
from jax.experimental.pallas import tpu as pltpu
from jax.experimental.pallas import tpu_sc as plsc

N = 10000
D = 128
H = 64
E = 320000
FILL = 0.5

NP = 10240
W = 32
K = 80
CPT = 132
EP = W * K * CPT
TAIL = (EP - E - NP) // 16
ET0 = E // 16
CH = 2000

_MESH = plsc.VectorSubcoreMesh(core_axis_name="c", subcore_axis_name="s")
_CP = pltpu.CompilerParams()
if "needs_layout_passes" in pltpu.CompilerParams.__dataclass_fields__:
    _CP = dataclasses.replace(_CP, needs_layout_passes=False)
if "use_tc_tiling_on_sc" in pltpu.CompilerParams.__dataclass_fields__:
    _CP = dataclasses.replace(_CP, use_tc_tiling_on_sc=False)


def _phase0(ei, w):
    @functools.partial(
        pl.kernel,
        out_type=(jax.ShapeDtypeStruct((EP,), jnp.float32),
                  jax.ShapeDtypeStruct((EP,), jnp.float32)),
        mesh=_MESH,
        compiler_params=_CP,
        scratch_types=[
            pltpu.VMEM((160, 64), jnp.float32),
            pltpu.VMEM((160, 64), jnp.float32),
            pltpu.VMEM((160, 64), jnp.float32),
            pltpu.VMEM((NP,), jnp.float32),
            pltpu.VMEM((CH,), jnp.int32),
            pltpu.VMEM((CH,), jnp.int32),
            pltpu.VMEM((CH,), jnp.float32),
            pltpu.VMEM((CH,), jnp.float32),
            pltpu.VMEM((640,), jnp.float32),
            pltpu.VMEM((640,), jnp.float32),
            pltpu.VMEM((10, 64), jnp.float32),
            pltpu.VMEM((160,), jnp.int32),
            pltpu.VMEM_SHARED((160, 64), jnp.float32),
            pltpu.VMEM_SHARED((160, 64), jnp.float32),
            pltpu.VMEM_SHARED((160, 64), jnp.float32),
            pltpu.VMEM_SHARED((NP,), jnp.float32),
        ],
    )
    def k(r_hbm, c_hbm, w_hbm, as_hbm, at_hbm, loc_deg, loc_cnt, loc_sw,
          loc_dinv, rbuf, cbuf, wbuf, abuf, diagbuf, dinvbuf, zbuf, idxref,
          sh_deg, sh_cnt, sh_sw, sh_dinv):
        cid = lax.axis_index("c")
        sid = lax.axis_index("s")
        z16 = jnp.zeros((16,), jnp.float32)
        iota = lax.iota(jnp.int32, 16)

        @pl.loop(0, 160)
        def _(i):
            for g in range(4):
                sl = pl.ds(g * 16, 16)
                loc_deg[i, sl] = z16
                loc_cnt[i, sl] = z16
                loc_sw[i, sl] = z16

        for p in range(10):
            for g in range(4):
                zbuf[p, pl.ds(g * 16, 16)] = z16
        for p in range(10):
            idxref[pl.ds(p * 16, 16)] = p * 16 + iota

        pltpu.sync_copy(zbuf, sh_deg.at[pl.ds(sid * 10, 10)])
        pltpu.sync_copy(zbuf, sh_cnt.at[pl.ds(sid * 10, 10)])
        pltpu.sync_copy(zbuf, sh_sw.at[pl.ds(sid * 10, 10)])
        plsc.subcore_barrier()

        base_e = sid * ET0

        @pl.loop(0, ET0 // CH)
        def _(ch):
            off = base_e + ch * CH
            pltpu.sync_copy(r_hbm.at[pl.ds(off, CH)], rbuf)
            pltpu.sync_copy(c_hbm.at[pl.ds(off, CH)], cbuf)
            pltpu.sync_copy(w_hbm.at[pl.ds(off, CH)], wbuf)

            @pl.loop(0, CH // 16)
            def _(g):
                sl = pl.ds(g * 16, 16)
                r = rbuf[sl]
                c = cbuf[sl]
                wv = wbuf[sl]
                m = r == c
                ew = jnp.where(m, 0.0, wv)
                dst = jnp.where(cid == 0, r, c)
                plsc.addupdate_scatter(loc_deg, [dst >> 6, dst & 63], ew)
                plsc.addupdate_scatter(
                    loc_cnt, [r >> 6, r & 63], jnp.where(m, 1.0, 0.0))
                plsc.addupdate_scatter(
                    loc_sw, [r >> 6, r & 63], jnp.where(m, wv, 0.0))

        pltpu.sync_copy(loc_deg, sh_deg.at[idxref], add=True)
        pltpu.sync_copy(loc_cnt, sh_cnt.at[idxref], add=True)
        pltpu.sync_copy(loc_sw, sh_sw.at[idxref], add=True)
        plsc.subcore_barrier()

        node0 = sid * 640
        pltpu.sync_copy(sh_deg.at[pl.ds(sid * 10, 10)],
                        loc_deg.at[pl.ds(0, 10)])
        pltpu.sync_copy(sh_cnt.at[pl.ds(sid * 10, 10)],
                        loc_cnt.at[pl.ds(0, 10)])
        pltpu.sync_copy(sh_sw.at[pl.ds(sid * 10, 10)],
                        loc_sw.at[pl.ds(0, 10)])
        for p in range(10):
            for g in range(4):
                sl = pl.ds(g * 16, 16)
                dg = loc_deg[p, sl]
                cn = loc_cnt[p, sl]
                sw = loc_sw[p, sl]
                has = cn > 0.0
                lw = jnp.where(has, sw / jnp.where(has, cn, 1.0), FILL)
                degt = dg + lw
                dinv = jnp.where(degt == 0.0, 0.0, 1.0 / degt)
                nid = node0 + p * 64 + g * 16 + iota
                o = pl.ds(p * 64 + g * 16, 16)
                dinvbuf[o] = dinv
                diagbuf[o] = jnp.where(nid < N, dinv * lw, 0.0)
        pltpu.sync_copy(dinvbuf, sh_dinv.at[pl.ds(node0, 640)])

        @pl.when(cid == 0)
        def _():
            pltpu.sync_copy(diagbuf, as_hbm.at[pl.ds(E + node0, 640)])

        @pl.when(cid == 1)
        def _():
            pltpu.sync_copy(diagbuf, at_hbm.at[pl.ds(E + node0, 640)])

        for p in range(TAIL // 16):
            diagbuf[pl.ds((p % 40) * 16, 16)] = z16
        tail0 = E + NP + sid * TAIL

        @pl.when(cid == 0)
        def _():
            pltpu.sync_copy(diagbuf.at[pl.ds(0, TAIL)],
                            as_hbm.at[pl.ds(tail0, TAIL)])

        @pl.when(cid == 1)
        def _():
            pltpu.sync_copy(diagbuf.at[pl.ds(0, TAIL)],
                            at_hbm.at[pl.ds(tail0, TAIL)])

        plsc.subcore_barrier()
        pltpu.sync_copy(sh_dinv, loc_dinv)

        @pl.loop(0, ET0 // CH)
        def _(ch):
            off = base_e + ch * CH
            pltpu.sync_copy(r_hbm.at[pl.ds(off, CH)], rbuf)
            pltpu.sync_copy(c_hbm.at[pl.ds(off, CH)], cbuf)
            pltpu.sync_copy(w_hbm.at[pl.ds(off, CH)], wbuf)

            @pl.loop(0, CH // 16)
            def _(g):
                sl = pl.ds(g * 16, 16)
                r = rbuf[sl]
                c = cbuf[sl]
                wv = wbuf[sl]
                ew = jnp.where(r == c, 0.0, wv)
                dst = jnp.where(cid == 0, r, c)
                abuf[sl] = plsc.load_gather(loc_dinv, [dst]) * ew

            @pl.when(cid == 0)
            def _():
                pltpu.sync_copy(abuf, as_hbm.at[pl.ds(off, CH)])

            @pl.when(cid == 1)
            def _():
                pltpu.sync_copy(abuf, at_hbm.at[pl.ds(off, CH)])

    return k(ei[0], ei[1], w)


def _conv(xL, xR, src3, dst3, a3, zeros):
    @functools.partial(
        pl.kernel,
        out_type=jax.ShapeDtypeStruct((2, 2, NP, H), jnp.float32),
        mesh=_MESH,
        compiler_params=_CP,
        scratch_types=[
            pltpu.VMEM((CPT, K), jnp.int32),
            pltpu.VMEM((CPT, K), jnp.int32),
            pltpu.VMEM((CPT, K), jnp.float32),
            pltpu.VMEM((K, H), jnp.float32),
            pltpu.VMEM_SHARED((NP, H), jnp.float32),
            pltpu.SemaphoreType.DMA,
            pltpu.SemaphoreType.DMA,
            pltpu.SemaphoreType.DMA,
            pltpu.SemaphoreType.DMA,
        ],
    )
    def k(xL_hbm, xR_hbm, src_hbm, dst_hbm, a_hbm, z_hbm, part_hbm,
          csrc, cdst, cav, rows, acc, sg0, sg1, ss0, ss1):
        cid = lax.axis_index("c")
        sid = lax.axis_index("s")
        wid = cid * 16 + sid
        r0 = sid * 640
        sg = (sg0, sg1)
        ss = (ss0, ss1)
        pltpu.sync_copy(src_hbm.at[wid], csrc)
        pltpu.sync_copy(dst_hbm.at[wid], cdst)
        pltpu.sync_copy(a_hbm.at[wid], cav)
        for f, x_hbm in ((0, xL_hbm), (1, xR_hbm)):
            pltpu.sync_copy(z_hbm.at[pl.ds(r0, 640)], acc.at[pl.ds(r0, 640)])
            plsc.subcore_barrier()

            @pl.loop(0, CPT)
            def _(ci):
                pltpu.async_copy(x_hbm.at[csrc.at[ci]], rows, sg[0]).wait()
                ci16 = lax.broadcast(ci, (16,))
                for j in range(K):
                    s = plsc.load_gather(
                        cav, [ci16, jnp.full((16,), j, jnp.int32)])
                    for g in range(4):
                        sl = pl.ds(g * 16, 16)
                        rows[j, sl] = rows[j, sl] * s

            plsc.subcore_barrier()
            pltpu.sync_copy(acc.at[pl.ds(r0, 640)],
                            part_hbm.at[cid, f, pl.ds(r0, 640)])

    return k(xL, xR, src3, dst3, a3, zeros)


def _combine(p):
    def body(p_ref, oL_ref, oR_ref):
        oL_ref[...] = p_ref[0, 0] + p_ref[1, 0]
        oR_ref[...] = p_ref[0, 1] + p_ref[1, 1]

    blk = pl.BlockSpec((1024, H), lambda i: (i, 0))
    return pl.pallas_call(
        body,
        grid=(10,),
        in_specs=[pl.BlockSpec((2, 2, 1024, H), lambda i: (0, 0, i, 0))],
        out_specs=[blk, blk],
        out_shape=[jax.ShapeDtypeStruct((NP, H), jnp.float32)] * 2,
    )(p)


def _final(wall, xsL, xsR, y1sL, y1sR, p2s, xtL, xtR, y1tL, y1tR, p2t):
    def body(w_ref, xsL_ref, xsR_ref, y1sL_ref, y1sR_ref, p2s_ref,
             xtL_ref, xtR_ref, y1tL_ref, y1tR_ref, p2t_ref,
             fsL_ref, fsR_ref, ftL_ref, ftR_ref):
        fsL_ref[...] = (w_ref[0] * xsL_ref[...] + w_ref[1] * y1sL_ref[...]
                        + w_ref[2] * (p2s_ref[0, 0] + p2s_ref[1, 0]))
        fsR_ref[...] = (w_ref[0] * xsR_ref[...] + w_ref[1] * y1sR_ref[...]
                        + w_ref[2] * (p2s_ref[0, 1] + p2s_ref[1, 1]))
        ftL_ref[...] = (w_ref[3] * xtL_ref[...] + w_ref[4] * y1tL_ref[...]
                        + w_ref[5] * (p2t_ref[0, 0] + p2t_ref[1, 0]))
        ftR_ref[...] = (w_ref[3] * xtR_ref[...] + w_ref[4] * y1tR_ref[...]
                        + w_ref[5] * (p2t_ref[0, 1] + p2t_ref[1, 1]))

    blk = pl.BlockSpec((1024, H), lambda i: (i, 0))
    blk4 = pl.BlockSpec((2, 2, 1024, H), lambda i: (0, 0, i, 0))
    return pl.pallas_call(
        body,
        grid=(10,),
        in_specs=[pl.BlockSpec(memory_space=pltpu.SMEM),
                  blk, blk, blk, blk, blk4, blk, blk, blk, blk, blk4],
        out_specs=[blk, blk, blk, blk],
        out_shape=[jax.ShapeDtypeStruct((NP, H), jnp.float32)] * 4,
    )(wall, xsL, xsR, y1sL, y1sR, p2s, xtL, xtR, y1tL, y1tR, p2t)


def kernel(x_s, x_t, edge_index, edge_weight, w_s, w_t):
    ei = edge_index.astype(jnp.int32)
    w = edge_weight.astype(jnp.float32)
    a_s, a_t = _phase0(ei, w)

    row, col = ei[0], ei[1]
    ar = jnp.arange(N, dtype=jnp.int32)
    zp = jnp.zeros((EP - E - N,), jnp.int32)
    src_s = jnp.concatenate([col, ar, zp]).reshape(W, CPT, K)
    dst_s = jnp.concatenate([row, ar, zp]).reshape(W, CPT, K)
    src_t = jnp.concatenate([row, ar, zp]).reshape(W, CPT, K)
    dst_t = jnp.concatenate([col, ar, zp]).reshape(W, CPT, K)
    a_s3 = a_s.reshape(W, CPT, K)
    a_t3 = a_t.reshape(W, CPT, K)

    zeros = jnp.zeros((NP, H), jnp.float32)
    pad = jnp.zeros((NP - N, H), jnp.float32)
    xsL = jnp.concatenate([x_s[:, :H], pad])
    xsR = jnp.concatenate([x_s[:, H:], pad])
    xtL = jnp.concatenate([x_t[:, :H], pad])
    xtR = jnp.concatenate([x_t[:, H:], pad])

    p1s = _conv(xsL, xsR, src_s, dst_s, a_s3, zeros)
    p1t = _conv(xtL, xtR, src_t, dst_t, a_t3, zeros)
    y1sL, y1sR = _combine(p1s)
    y1tL, y1tR = _combine(p1t)
    p2s = _conv(y1sL, y1sR, src_s, dst_s, a_s3, zeros)
    p2t = _conv(y1tL, y1tR, src_t, dst_t, a_t3, zeros)

    wall = jnp.concatenate([w_s.astype(jnp.float32)[:, 0],
                            w_t.astype(jnp.float32)[:, 0]])
    fsL, fsR, ftL, ftR = _final(wall, xsL, xsR, y1sL, y1sR, p2s,
                                xtL, xtR, y1tL, y1tR, p2t)
    return jnp.concatenate([fsL[:N], fsR[:N], ftL[:N], ftR[:N]], axis=1)


# speedup vs baseline: 1.6228x; 1.6228x over previous
import dataclasses
import functools

import jax
import jax.numpy as jnp
from jax import lax
from jax.experimental import pallas as pl
from jax.experimental.pallas import tpu as pltpu
from jax.experimental.pallas import tpu_sc as plsc

N = 10000
D = 128
H = 64
E = 320000
FILL = 0.5

NP = 10240
W = 32
K = 80
CPT = 129
EP = W * K * CPT
ET0 = E // 16
CH = 2000

_MESH = plsc.VectorSubcoreMesh(core_axis_name="c", subcore_axis_name="s")
_CP = pltpu.CompilerParams()
if "needs_layout_passes" in pltpu.CompilerParams.__dataclass_fields__:
    _CP = dataclasses.replace(_CP, needs_layout_passes=False)
if "use_tc_tiling_on_sc" in pltpu.CompilerParams.__dataclass_fields__:
    _CP = dataclasses.replace(_CP, use_tc_tiling_on_sc=False)


def _phase0(ei, w):
    @functools.partial(
        pl.kernel,
        out_type=(jax.ShapeDtypeStruct((EP,), jnp.float32),
                  jax.ShapeDtypeStruct((EP,), jnp.float32)),
        mesh=_MESH,
        compiler_params=_CP,
        scratch_types=[
            pltpu.VMEM((160, 64), jnp.float32),
            pltpu.VMEM((160, 64), jnp.float32),
            pltpu.VMEM((160, 64), jnp.float32),
            pltpu.VMEM((NP,), jnp.float32),
            pltpu.VMEM((CH,), jnp.int32),
            pltpu.VMEM((CH,), jnp.int32),
            pltpu.VMEM((CH,), jnp.float32),
            pltpu.VMEM((CH,), jnp.float32),
            pltpu.VMEM((640,), jnp.float32),
            pltpu.VMEM((640,), jnp.float32),
            pltpu.VMEM((10, 64), jnp.float32),
            pltpu.VMEM((160,), jnp.int32),
            pltpu.VMEM_SHARED((160, 64), jnp.float32),
            pltpu.VMEM_SHARED((160, 64), jnp.float32),
            pltpu.VMEM_SHARED((160, 64), jnp.float32),
            pltpu.VMEM_SHARED((NP,), jnp.float32),
        ],
    )
    def k(r_hbm, c_hbm, w_hbm, as_hbm, at_hbm, loc_deg, loc_cnt, loc_sw,
          loc_dinv, rbuf, cbuf, wbuf, abuf, diagbuf, dinvbuf, zbuf, idxref,
          sh_deg, sh_cnt, sh_sw, sh_dinv):
        cid = lax.axis_index("c")
        sid = lax.axis_index("s")
        z16 = jnp.zeros((16,), jnp.float32)
        iota = lax.iota(jnp.int32, 16)

        @pl.loop(0, 160)
        def _(i):
            for g in range(4):
                sl = pl.ds(g * 16, 16)
                loc_deg[i, sl] = z16
                loc_cnt[i, sl] = z16
                loc_sw[i, sl] = z16

        for p in range(10):
            for g in range(4):
                zbuf[p, pl.ds(g * 16, 16)] = z16
        for p in range(10):
            idxref[pl.ds(p * 16, 16)] = p * 16 + iota

        pltpu.sync_copy(zbuf, sh_deg.at[pl.ds(sid * 10, 10)])
        pltpu.sync_copy(zbuf, sh_cnt.at[pl.ds(sid * 10, 10)])
        pltpu.sync_copy(zbuf, sh_sw.at[pl.ds(sid * 10, 10)])
        plsc.subcore_barrier()

        base_e = sid * ET0

        @pl.loop(0, ET0 // CH)
        def _(ch):
            off = base_e + ch * CH
            pltpu.sync_copy(r_hbm.at[pl.ds(off, CH)], rbuf)
            pltpu.sync_copy(c_hbm.at[pl.ds(off, CH)], cbuf)
            pltpu.sync_copy(w_hbm.at[pl.ds(off, CH)], wbuf)

            @pl.loop(0, CH // 16)
            def _(g):
                sl = pl.ds(g * 16, 16)
                r = rbuf[sl]
                c = cbuf[sl]
                wv = wbuf[sl]
                m = r == c
                ew = jnp.where(m, 0.0, wv)
                dst = jnp.where(cid == 0, r, c)
                plsc.addupdate_scatter(loc_deg, [dst >> 6, dst & 63], ew)
                plsc.addupdate_scatter(
                    loc_cnt, [r >> 6, r & 63], jnp.where(m, 1.0, 0.0))
                plsc.addupdate_scatter(
                    loc_sw, [r >> 6, r & 63], jnp.where(m, wv, 0.0))

        pltpu.sync_copy(loc_deg, sh_deg.at[idxref], add=True)
        pltpu.sync_copy(loc_cnt, sh_cnt.at[idxref], add=True)
        pltpu.sync_copy(loc_sw, sh_sw.at[idxref], add=True)
        plsc.subcore_barrier()

        node0 = sid * 640
        pltpu.sync_copy(sh_deg.at[pl.ds(sid * 10, 10)],
                        loc_deg.at[pl.ds(0, 10)])
        pltpu.sync_copy(sh_cnt.at[pl.ds(sid * 10, 10)],
                        loc_cnt.at[pl.ds(0, 10)])
        pltpu.sync_copy(sh_sw.at[pl.ds(sid * 10, 10)],
                        loc_sw.at[pl.ds(0, 10)])
        for p in range(10):
            for g in range(4):
                sl = pl.ds(g * 16, 16)
                dg = loc_deg[p, sl]
                cn = loc_cnt[p, sl]
                sw = loc_sw[p, sl]
                has = cn > 0.0
                lw = jnp.where(has, sw / jnp.where(has, cn, 1.0), FILL)
                degt = dg + lw
                dinv = jnp.where(degt == 0.0, 0.0, 1.0 / degt)
                nid = node0 + p * 64 + g * 16 + iota
                o = pl.ds(p * 64 + g * 16, 16)
                dinvbuf[o] = dinv
                diagbuf[o] = jnp.where(nid < N, dinv * lw, 0.0)
        pltpu.sync_copy(dinvbuf, sh_dinv.at[pl.ds(node0, 640)])

        @pl.when(cid == 0)
        def _():
            pltpu.sync_copy(diagbuf, as_hbm.at[pl.ds(E + node0, 640)])

        @pl.when(cid == 1)
        def _():
            pltpu.sync_copy(diagbuf, at_hbm.at[pl.ds(E + node0, 640)])

        plsc.subcore_barrier()
        pltpu.sync_copy(sh_dinv, loc_dinv)

        @pl.loop(0, ET0 // CH)
        def _(ch):
            off = base_e + ch * CH
            pltpu.sync_copy(r_hbm.at[pl.ds(off, CH)], rbuf)
            pltpu.sync_copy(c_hbm.at[pl.ds(off, CH)], cbuf)
            pltpu.sync_copy(w_hbm.at[pl.ds(off, CH)], wbuf)

            @pl.loop(0, CH // 16)
            def _(g):
                sl = pl.ds(g * 16, 16)
                r = rbuf[sl]
                c = cbuf[sl]
                wv = wbuf[sl]
                ew = jnp.where(r == c, 0.0, wv)
                dst = jnp.where(cid == 0, r, c)
                abuf[sl] = plsc.load_gather(loc_dinv, [dst]) * ew

            @pl.when(cid == 0)
            def _():
                pltpu.sync_copy(abuf, as_hbm.at[pl.ds(off, CH)])

            @pl.when(cid == 1)
            def _():
                pltpu.sync_copy(abuf, at_hbm.at[pl.ds(off, CH)])

    return k(ei[0], ei[1], w)


def _conv(xL, xR, src3, dst3, a3, zeros):
    @functools.partial(
        pl.kernel,
        out_type=jax.ShapeDtypeStruct((2, 2, NP, H), jnp.float32),
        mesh=_MESH,
        compiler_params=_CP,
        scratch_types=[
            pltpu.VMEM((CPT, K), jnp.int32),
            pltpu.VMEM((CPT, K), jnp.int32),
            pltpu.VMEM((CPT, K), jnp.float32),
            pltpu.VMEM((K, H), jnp.float32),
            pltpu.VMEM_SHARED((NP, H), jnp.float32),
            pltpu.SemaphoreType.DMA,
        ],
    )
    def k(xL_hbm, xR_hbm, src_hbm, dst_hbm, a_hbm, z_hbm, part_hbm,
          csrc, cdst, cav, rows, acc, sem):
        cid = lax.axis_index("c")
        sid = lax.axis_index("s")
        wid = cid * 16 + sid
        r0 = sid * 640
        pltpu.sync_copy(src_hbm.at[wid], csrc)
        pltpu.sync_copy(dst_hbm.at[wid], cdst)
        pltpu.sync_copy(a_hbm.at[wid], cav)
        for f, x_hbm in ((0, xL_hbm), (1, xR_hbm)):
            pltpu.sync_copy(z_hbm.at[pl.ds(r0, 640)], acc.at[pl.ds(r0, 640)])
            plsc.subcore_barrier()

            @pl.loop(0, CPT)
            def _(ci):
                pltpu.async_copy(x_hbm.at[csrc.at[ci]], rows, sem).wait()
                ci16 = lax.broadcast(ci, (16,))
                for j in range(K):
                    s = plsc.load_gather(
                        cav, [ci16, jnp.full((16,), j, jnp.int32)])
                    for g in range(4):
                        sl = pl.ds(g * 16, 16)
                        rows[j, sl] = rows[j, sl] * s
                pltpu.sync_copy(rows, acc.at[cdst.at[ci]], add=True)

            plsc.subcore_barrier()
            pltpu.sync_copy(acc.at[pl.ds(r0, 640)],
                            part_hbm.at[cid, f, pl.ds(r0, 640)])

    return k(xL, xR, src3, dst3, a3, zeros)


def _combine(p):
    def body(p_ref, oL_ref, oR_ref):
        oL_ref[...] = p_ref[0, 0] + p_ref[1, 0]
        oR_ref[...] = p_ref[0, 1] + p_ref[1, 1]

    blk = pl.BlockSpec((1024, H), lambda i: (i, 0))
    return pl.pallas_call(
        body,
        grid=(10,),
        in_specs=[pl.BlockSpec((2, 2, 1024, H), lambda i: (0, 0, i, 0))],
        out_specs=[blk, blk],
        out_shape=[jax.ShapeDtypeStruct((NP, H), jnp.float32)] * 2,
    )(p)


def _final(wall, xsL, xsR, y1sL, y1sR, p2s, xtL, xtR, y1tL, y1tR, p2t):
    def body(w_ref, xsL_ref, xsR_ref, y1sL_ref, y1sR_ref, p2s_ref,
             xtL_ref, xtR_ref, y1tL_ref, y1tR_ref, p2t_ref,
             fsL_ref, fsR_ref, ftL_ref, ftR_ref):
        fsL_ref[...] = (w_ref[0] * xsL_ref[...] + w_ref[1] * y1sL_ref[...]
                        + w_ref[2] * (p2s_ref[0, 0] + p2s_ref[1, 0]))
        fsR_ref[...] = (w_ref[0] * xsR_ref[...] + w_ref[1] * y1sR_ref[...]
                        + w_ref[2] * (p2s_ref[0, 1] + p2s_ref[1, 1]))
        ftL_ref[...] = (w_ref[3] * xtL_ref[...] + w_ref[4] * y1tL_ref[...]
                        + w_ref[5] * (p2t_ref[0, 0] + p2t_ref[1, 0]))
        ftR_ref[...] = (w_ref[3] * xtR_ref[...] + w_ref[4] * y1tR_ref[...]
                        + w_ref[5] * (p2t_ref[0, 1] + p2t_ref[1, 1]))

    blk = pl.BlockSpec((1024, H), lambda i: (i, 0))
    blk4 = pl.BlockSpec((2, 2, 1024, H), lambda i: (0, 0, i, 0))
    return pl.pallas_call(
        body,
        grid=(10,),
        in_specs=[pl.BlockSpec(memory_space=pltpu.SMEM),
                  blk, blk, blk, blk, blk4, blk, blk, blk, blk, blk4],
        out_specs=[blk, blk, blk, blk],
        out_shape=[jax.ShapeDtypeStruct((NP, H), jnp.float32)] * 4,
    )(wall, xsL, xsR, y1sL, y1sR, p2s, xtL, xtR, y1tL, y1tR, p2t)


def kernel(x_s, x_t, edge_index, edge_weight, w_s, w_t):
    ei = edge_index.astype(jnp.int32)
    w = edge_weight.astype(jnp.float32)
    a_s, a_t = _phase0(ei, w)

    row, col = ei[0], ei[1]
    ar = jnp.arange(N, dtype=jnp.int32)
    zp = jnp.zeros((EP - E - N,), jnp.int32)
    src_s = jnp.concatenate([col, ar, zp]).reshape(W, CPT, K)
    dst_s = jnp.concatenate([row, ar, zp]).reshape(W, CPT, K)
    src_t = jnp.concatenate([row, ar, zp]).reshape(W, CPT, K)
    dst_t = jnp.concatenate([col, ar, zp]).reshape(W, CPT, K)
    a_s3 = a_s.reshape(W, CPT, K)
    a_t3 = a_t.reshape(W, CPT, K)

    zeros = jnp.zeros((NP, H), jnp.float32)
    pad = jnp.zeros((NP - N, H), jnp.float32)
    xsL = jnp.concatenate([x_s[:, :H], pad])
    xsR = jnp.concatenate([x_s[:, H:], pad])
    xtL = jnp.concatenate([x_t[:, :H], pad])
    xtR = jnp.concatenate([x_t[:, H:], pad])

    p1s = _conv(xsL, xsR, src_s, dst_s, a_s3, zeros)
    p1t = _conv(xtL, xtR, src_t, dst_t, a_t3, zeros)
    y1sL, y1sR = _combine(p1s)
    y1tL, y1tR = _combine(p1t)
    p2s = _conv(y1sL, y1sR, src_s, dst_s, a_s3, zeros)
    p2t = _conv(y1tL, y1tR, src_t, dst_t, a_t3, zeros)

    wall = jnp.concatenate([w_s.astype(jnp.float32)[:, 0],
                            w_t.astype(jnp.float32)[:, 0]])
    fsL, fsR, ftL, ftR = _final(wall, xsL, xsR, y1sL, y1sR, p2s,
                                xtL, xtR, y1tL, y1tR, p2t)
    return jnp.concatenate([fsL[:N], fsR[:N], ftL[:N], ftR[:N]], axis=1)


# speedup vs baseline: 1.8171x; 1.1197x over previous
import dataclasses
import functools

import jax
import jax.numpy as jnp
from jax import lax
from jax.experimental import pallas as pl
from jax.experimental.pallas import tpu as pltpu
from jax.experimental.pallas import tpu_sc as plsc

N = 10000
D = 128
H = 64
E = 320000
FILL = 0.5

NP = 10240
W = 32
K = 120
CPT = 87
EP = W * K * CPT
TAIL = (EP - E - NP) // 16
ET0 = E // 16
CH = 2000

_MESH = plsc.VectorSubcoreMesh(core_axis_name="c", subcore_axis_name="s")
_CP = pltpu.CompilerParams()
if "needs_layout_passes" in pltpu.CompilerParams.__dataclass_fields__:
    _CP = dataclasses.replace(_CP, needs_layout_passes=False)
if "use_tc_tiling_on_sc" in pltpu.CompilerParams.__dataclass_fields__:
    _CP = dataclasses.replace(_CP, use_tc_tiling_on_sc=False)


def _phase0(ei, w):
    @functools.partial(
        pl.kernel,
        out_type=(jax.ShapeDtypeStruct((EP,), jnp.float32),
                  jax.ShapeDtypeStruct((EP,), jnp.float32)),
        mesh=_MESH,
        compiler_params=_CP,
        scratch_types=[
            pltpu.VMEM((160, 64), jnp.float32),
            pltpu.VMEM((160, 64), jnp.float32),
            pltpu.VMEM((160, 64), jnp.float32),
            pltpu.VMEM((NP,), jnp.float32),
            pltpu.VMEM((CH,), jnp.int32),
            pltpu.VMEM((CH,), jnp.int32),
            pltpu.VMEM((CH,), jnp.float32),
            pltpu.VMEM((CH,), jnp.float32),
            pltpu.VMEM((640,), jnp.float32),
            pltpu.VMEM((640,), jnp.float32),
            pltpu.VMEM((10, 64), jnp.float32),
            pltpu.VMEM((160,), jnp.int32),
            pltpu.VMEM_SHARED((160, 64), jnp.float32),
            pltpu.VMEM_SHARED((160, 64), jnp.float32),
            pltpu.VMEM_SHARED((160, 64), jnp.float32),
            pltpu.VMEM_SHARED((NP,), jnp.float32),
        ],
    )
    def k(r_hbm, c_hbm, w_hbm, as_hbm, at_hbm, loc_deg, loc_cnt, loc_sw,
          loc_dinv, rbuf, cbuf, wbuf, abuf, diagbuf, dinvbuf, zbuf, idxref,
          sh_deg, sh_cnt, sh_sw, sh_dinv):
        cid = lax.axis_index("c")
        sid = lax.axis_index("s")
        z16 = jnp.zeros((16,), jnp.float32)
        iota = lax.iota(jnp.int32, 16)

        @pl.loop(0, 160)
        def _(i):
            for g in range(4):
                sl = pl.ds(g * 16, 16)
                loc_deg[i, sl] = z16
                loc_cnt[i, sl] = z16
                loc_sw[i, sl] = z16

        for p in range(10):
            for g in range(4):
                zbuf[p, pl.ds(g * 16, 16)] = z16
        for p in range(10):
            idxref[pl.ds(p * 16, 16)] = p * 16 + iota

        pltpu.sync_copy(zbuf, sh_deg.at[pl.ds(sid * 10, 10)])
        pltpu.sync_copy(zbuf, sh_cnt.at[pl.ds(sid * 10, 10)])
        pltpu.sync_copy(zbuf, sh_sw.at[pl.ds(sid * 10, 10)])
        plsc.subcore_barrier()

        base_e = sid * ET0

        @pl.loop(0, ET0 // CH)
        def _(ch):
            off = base_e + ch * CH
            pltpu.sync_copy(r_hbm.at[pl.ds(off, CH)], rbuf)
            pltpu.sync_copy(c_hbm.at[pl.ds(off, CH)], cbuf)
            pltpu.sync_copy(w_hbm.at[pl.ds(off, CH)], wbuf)

            @pl.loop(0, CH // 16)
            def _(g):
                sl = pl.ds(g * 16, 16)
                r = rbuf[sl]
                c = cbuf[sl]
                wv = wbuf[sl]
                m = r == c
                ew = jnp.where(m, 0.0, wv)
                dst = jnp.where(cid == 0, r, c)
                plsc.addupdate_scatter(loc_deg, [dst >> 6, dst & 63], ew)
                plsc.addupdate_scatter(
                    loc_cnt, [r >> 6, r & 63], jnp.where(m, 1.0, 0.0))
                plsc.addupdate_scatter(
                    loc_sw, [r >> 6, r & 63], jnp.where(m, wv, 0.0))

        pltpu.sync_copy(loc_deg, sh_deg.at[idxref], add=True)
        pltpu.sync_copy(loc_cnt, sh_cnt.at[idxref], add=True)
        pltpu.sync_copy(loc_sw, sh_sw.at[idxref], add=True)
        plsc.subcore_barrier()

        node0 = sid * 640
        pltpu.sync_copy(sh_deg.at[pl.ds(sid * 10, 10)],
                        loc_deg.at[pl.ds(0, 10)])
        pltpu.sync_copy(sh_cnt.at[pl.ds(sid * 10, 10)],
                        loc_cnt.at[pl.ds(0, 10)])
        pltpu.sync_copy(sh_sw.at[pl.ds(sid * 10, 10)],
                        loc_sw.at[pl.ds(0, 10)])
        for p in range(10):
            for g in range(4):
                sl = pl.ds(g * 16, 16)
                dg = loc_deg[p, sl]
                cn = loc_cnt[p, sl]
                sw = loc_sw[p, sl]
                has = cn > 0.0
                lw = jnp.where(has, sw / jnp.where(has, cn, 1.0), FILL)
                degt = dg + lw
                dinv = jnp.where(degt == 0.0, 0.0, 1.0 / degt)
                nid = node0 + p * 64 + g * 16 + iota
                o = pl.ds(p * 64 + g * 16, 16)
                dinvbuf[o] = dinv
                diagbuf[o] = jnp.where(nid < N, dinv * lw, 0.0)
        pltpu.sync_copy(dinvbuf, sh_dinv.at[pl.ds(node0, 640)])

        @pl.when(cid == 0)
        def _():
            pltpu.sync_copy(diagbuf, as_hbm.at[pl.ds(E + node0, 640)])

        @pl.when(cid == 1)
        def _():
            pltpu.sync_copy(diagbuf, at_hbm.at[pl.ds(E + node0, 640)])

        for p in range(TAIL // 16):
            diagbuf[pl.ds(p * 16, 16)] = z16
        tail0 = E + NP + sid * TAIL

        @pl.when(cid == 0)
        def _():
            pltpu.sync_copy(diagbuf.at[pl.ds(0, TAIL)],
                            as_hbm.at[pl.ds(tail0, TAIL)])

        @pl.when(cid == 1)
        def _():
            pltpu.sync_copy(diagbuf.at[pl.ds(0, TAIL)],
                            at_hbm.at[pl.ds(tail0, TAIL)])

        plsc.subcore_barrier()
        pltpu.sync_copy(sh_dinv, loc_dinv)

        @pl.loop(0, ET0 // CH)
        def _(ch):
            off = base_e + ch * CH
            pltpu.sync_copy(r_hbm.at[pl.ds(off, CH)], rbuf)
            pltpu.sync_copy(c_hbm.at[pl.ds(off, CH)], cbuf)
            pltpu.sync_copy(w_hbm.at[pl.ds(off, CH)], wbuf)

            @pl.loop(0, CH // 16)
            def _(g):
                sl = pl.ds(g * 16, 16)
                r = rbuf[sl]
                c = cbuf[sl]
                wv = wbuf[sl]
                ew = jnp.where(r == c, 0.0, wv)
                dst = jnp.where(cid == 0, r, c)
                abuf[sl] = plsc.load_gather(loc_dinv, [dst]) * ew

            @pl.when(cid == 0)
            def _():
                pltpu.sync_copy(abuf, as_hbm.at[pl.ds(off, CH)])

            @pl.when(cid == 1)
            def _():
                pltpu.sync_copy(abuf, at_hbm.at[pl.ds(off, CH)])

    return k(ei[0], ei[1], w)


def _conv(xL, xR, src3, dst3, a3, zeros):
    @functools.partial(
        pl.kernel,
        out_type=jax.ShapeDtypeStruct((2, 2, NP, H), jnp.float32),
        mesh=_MESH,
        compiler_params=_CP,
        scratch_types=[
            pltpu.VMEM((CPT, K), jnp.int32),
            pltpu.VMEM((CPT, K), jnp.int32),
            pltpu.VMEM((CPT, K), jnp.float32),
            pltpu.VMEM((K, H), jnp.float32),
            pltpu.VMEM_SHARED((NP, H), jnp.float32),
            pltpu.SemaphoreType.DMA,
        ],
    )
    def k(xL_hbm, xR_hbm, src_hbm, dst_hbm, a_hbm, z_hbm, part_hbm,
          csrc, cdst, cav, rows, acc, sem):
        cid = lax.axis_index("c")
        sid = lax.axis_index("s")
        wid = cid * 16 + sid
        r0 = sid * 640
        pltpu.sync_copy(src_hbm.at[wid], csrc)
        pltpu.sync_copy(dst_hbm.at[wid], cdst)
        pltpu.sync_copy(a_hbm.at[wid], cav)
        for f, x_hbm in ((0, xL_hbm), (1, xR_hbm)):
            pltpu.sync_copy(z_hbm.at[pl.ds(r0, 640)], acc.at[pl.ds(r0, 640)])
            plsc.subcore_barrier()

            @pl.loop(0, CPT)
            def _(ci):
                pltpu.async_copy(x_hbm.at[csrc.at[ci]], rows, sem).wait()
                ci16 = lax.broadcast(ci, (16,))
                for j in range(K):
                    s = plsc.load_gather(
                        cav, [ci16, jnp.full((16,), j, jnp.int32)])
                    for g in range(4):
                        sl = pl.ds(g * 16, 16)
                        rows[j, sl] = rows[j, sl] * s
                pltpu.sync_copy(rows, acc.at[cdst.at[ci]], add=True)

            plsc.subcore_barrier()
            pltpu.sync_copy(acc.at[pl.ds(r0, 640)],
                            part_hbm.at[cid, f, pl.ds(r0, 640)])

    return k(xL, xR, src3, dst3, a3, zeros)


def _combine(p):
    def body(p_ref, oL_ref, oR_ref):
        oL_ref[...] = p_ref[0, 0] + p_ref[1, 0]
        oR_ref[...] = p_ref[0, 1] + p_ref[1, 1]

    blk = pl.BlockSpec((1024, H), lambda i: (i, 0))
    return pl.pallas_call(
        body,
        grid=(10,),
        in_specs=[pl.BlockSpec((2, 2, 1024, H), lambda i: (0, 0, i, 0))],
        out_specs=[blk, blk],
        out_shape=[jax.ShapeDtypeStruct((NP, H), jnp.float32)] * 2,
    )(p)


def _final(wall, xsL, xsR, y1sL, y1sR, p2s, xtL, xtR, y1tL, y1tR, p2t):
    def body(w_ref, xsL_ref, xsR_ref, y1sL_ref, y1sR_ref, p2s_ref,
             xtL_ref, xtR_ref, y1tL_ref, y1tR_ref, p2t_ref,
             fsL_ref, fsR_ref, ftL_ref, ftR_ref):
        fsL_ref[...] = (w_ref[0] * xsL_ref[...] + w_ref[1] * y1sL_ref[...]
                        + w_ref[2] * (p2s_ref[0, 0] + p2s_ref[1, 0]))
        fsR_ref[...] = (w_ref[0] * xsR_ref[...] + w_ref[1] * y1sR_ref[...]
                        + w_ref[2] * (p2s_ref[0, 1] + p2s_ref[1, 1]))
        ftL_ref[...] = (w_ref[3] * xtL_ref[...] + w_ref[4] * y1tL_ref[...]
                        + w_ref[5] * (p2t_ref[0, 0] + p2t_ref[1, 0]))
        ftR_ref[...] = (w_ref[3] * xtR_ref[...] + w_ref[4] * y1tR_ref[...]
                        + w_ref[5] * (p2t_ref[0, 1] + p2t_ref[1, 1]))

    blk = pl.BlockSpec((1024, H), lambda i: (i, 0))
    blk4 = pl.BlockSpec((2, 2, 1024, H), lambda i: (0, 0, i, 0))
    return pl.pallas_call(
        body,
        grid=(10,),
        in_specs=[pl.BlockSpec(memory_space=pltpu.SMEM),
                  blk, blk, blk, blk, blk4, blk, blk, blk, blk, blk4],
        out_specs=[blk, blk, blk, blk],
        out_shape=[jax.ShapeDtypeStruct((NP, H), jnp.float32)] * 4,
    )(wall, xsL, xsR, y1sL, y1sR, p2s, xtL, xtR, y1tL, y1tR, p2t)


def kernel(x_s, x_t, edge_index, edge_weight, w_s, w_t):
    ei = edge_index.astype(jnp.int32)
    w = edge_weight.astype(jnp.float32)
    a_s, a_t = _phase0(ei, w)

    row, col = ei[0], ei[1]
    ar = jnp.arange(N, dtype=jnp.int32)
    zp = (jnp.arange(EP - E - N, dtype=jnp.int32) * 97) % N
    src_s = jnp.concatenate([col, ar, zp]).reshape(W, CPT, K)
    dst_s = jnp.concatenate([row, ar, zp]).reshape(W, CPT, K)
    src_t = jnp.concatenate([row, ar, zp]).reshape(W, CPT, K)
    dst_t = jnp.concatenate([col, ar, zp]).reshape(W, CPT, K)
    a_s3 = a_s.reshape(W, CPT, K)
    a_t3 = a_t.reshape(W, CPT, K)

    zeros = jnp.zeros((NP, H), jnp.float32)
    pad = jnp.zeros((NP - N, H), jnp.float32)
    xsL = jnp.concatenate([x_s[:, :H], pad])
    xsR = jnp.concatenate([x_s[:, H:], pad])
    xtL = jnp.concatenate([x_t[:, :H], pad])
    xtR = jnp.concatenate([x_t[:, H:], pad])

    p1s = _conv(xsL, xsR, src_s, dst_s, a_s3, zeros)
    p1t = _conv(xtL, xtR, src_t, dst_t, a_t3, zeros)
    y1sL, y1sR = _combine(p1s)
    y1tL, y1tR = _combine(p1t)
    p2s = _conv(y1sL, y1sR, src_s, dst_s, a_s3, zeros)
    p2t = _conv(y1tL, y1tR, src_t, dst_t, a_t3, zeros)

    wall = jnp.concatenate([w_s.astype(jnp.float32)[:, 0],
                            w_t.astype(jnp.float32)[:, 0]])
    fsL, fsR, ftL, ftR = _final(wall, xsL, xsR, y1sL, y1sR, p2s,
                                xtL, xtR, y1tL, y1tR, p2t)
    return jnp.concatenate([fsL[:N], fsR[:N], ftL[:N], ftR[:N]], axis=1)


# speedup vs baseline: 1.8338x; 1.0092x over previous
import dataclasses
import functools

import jax
import jax.numpy as jnp
from jax import lax
from jax.experimental import pallas as pl
from jax.experimental.pallas import tpu as pltpu
from jax.experimental.pallas import tpu_sc as plsc

N = 10000
D = 128
H = 64
E = 320000
FILL = 0.5

NP = 10240
W = 32
K = 128
CPT = 82
EP = W * K * CPT
TAIL = (EP - E - NP) // 16
ET0 = E // 16
CH = 2000

_MESH = plsc.VectorSubcoreMesh(core_axis_name="c", subcore_axis_name="s")
_CP = pltpu.CompilerParams()
if "needs_layout_passes" in pltpu.CompilerParams.__dataclass_fields__:
    _CP = dataclasses.replace(_CP, needs_layout_passes=False)
if "use_tc_tiling_on_sc" in pltpu.CompilerParams.__dataclass_fields__:
    _CP = dataclasses.replace(_CP, use_tc_tiling_on_sc=False)


def _phase0(ei, w):
    @functools.partial(
        pl.kernel,
        out_type=(jax.ShapeDtypeStruct((EP,), jnp.float32),
                  jax.ShapeDtypeStruct((EP,), jnp.float32)),
        mesh=_MESH,
        compiler_params=_CP,
        scratch_types=[
            pltpu.VMEM((160, 64), jnp.float32),
            pltpu.VMEM((160, 64), jnp.float32),
            pltpu.VMEM((160, 64), jnp.float32),
            pltpu.VMEM((NP,), jnp.float32),
            pltpu.VMEM((CH,), jnp.int32),
            pltpu.VMEM((CH,), jnp.int32),
            pltpu.VMEM((CH,), jnp.float32),
            pltpu.VMEM((CH,), jnp.float32),
            pltpu.VMEM((640,), jnp.float32),
            pltpu.VMEM((640,), jnp.float32),
            pltpu.VMEM((10, 64), jnp.float32),
            pltpu.VMEM((160,), jnp.int32),
            pltpu.VMEM_SHARED((160, 64), jnp.float32),
            pltpu.VMEM_SHARED((160, 64), jnp.float32),
            pltpu.VMEM_SHARED((160, 64), jnp.float32),
            pltpu.VMEM_SHARED((NP,), jnp.float32),
        ],
    )
    def k(r_hbm, c_hbm, w_hbm, as_hbm, at_hbm, loc_deg, loc_cnt, loc_sw,
          loc_dinv, rbuf, cbuf, wbuf, abuf, diagbuf, dinvbuf, zbuf, idxref,
          sh_deg, sh_cnt, sh_sw, sh_dinv):
        cid = lax.axis_index("c")
        sid = lax.axis_index("s")
        z16 = jnp.zeros((16,), jnp.float32)
        iota = lax.iota(jnp.int32, 16)

        @pl.loop(0, 160)
        def _(i):
            for g in range(4):
                sl = pl.ds(g * 16, 16)
                loc_deg[i, sl] = z16
                loc_cnt[i, sl] = z16
                loc_sw[i, sl] = z16

        for p in range(10):
            for g in range(4):
                zbuf[p, pl.ds(g * 16, 16)] = z16
        for p in range(10):
            idxref[pl.ds(p * 16, 16)] = p * 16 + iota

        pltpu.sync_copy(zbuf, sh_deg.at[pl.ds(sid * 10, 10)])
        pltpu.sync_copy(zbuf, sh_cnt.at[pl.ds(sid * 10, 10)])
        pltpu.sync_copy(zbuf, sh_sw.at[pl.ds(sid * 10, 10)])
        plsc.subcore_barrier()

        base_e = sid * ET0

        @pl.loop(0, ET0 // CH)
        def _(ch):
            off = base_e + ch * CH
            pltpu.sync_copy(r_hbm.at[pl.ds(off, CH)], rbuf)
            pltpu.sync_copy(c_hbm.at[pl.ds(off, CH)], cbuf)
            pltpu.sync_copy(w_hbm.at[pl.ds(off, CH)], wbuf)

            @pl.loop(0, CH // 16)
            def _(g):
                sl = pl.ds(g * 16, 16)
                r = rbuf[sl]
                c = cbuf[sl]
                wv = wbuf[sl]
                m = r == c
                ew = jnp.where(m, 0.0, wv)
                dst = jnp.where(cid == 0, r, c)
                plsc.addupdate_scatter(loc_deg, [dst >> 6, dst & 63], ew)
                plsc.addupdate_scatter(
                    loc_cnt, [r >> 6, r & 63], jnp.where(m, 1.0, 0.0))
                plsc.addupdate_scatter(
                    loc_sw, [r >> 6, r & 63], jnp.where(m, wv, 0.0))

        pltpu.sync_copy(loc_deg, sh_deg.at[idxref], add=True)
        pltpu.sync_copy(loc_cnt, sh_cnt.at[idxref], add=True)
        pltpu.sync_copy(loc_sw, sh_sw.at[idxref], add=True)
        plsc.subcore_barrier()

        node0 = sid * 640
        pltpu.sync_copy(sh_deg.at[pl.ds(sid * 10, 10)],
                        loc_deg.at[pl.ds(0, 10)])
        pltpu.sync_copy(sh_cnt.at[pl.ds(sid * 10, 10)],
                        loc_cnt.at[pl.ds(0, 10)])
        pltpu.sync_copy(sh_sw.at[pl.ds(sid * 10, 10)],
                        loc_sw.at[pl.ds(0, 10)])
        for p in range(10):
            for g in range(4):
                sl = pl.ds(g * 16, 16)
                dg = loc_deg[p, sl]
                cn = loc_cnt[p, sl]
                sw = loc_sw[p, sl]
                has = cn > 0.0
                lw = jnp.where(has, sw / jnp.where(has, cn, 1.0), FILL)
                degt = dg + lw
                dinv = jnp.where(degt == 0.0, 0.0, 1.0 / degt)
                nid = node0 + p * 64 + g * 16 + iota
                o = pl.ds(p * 64 + g * 16, 16)
                dinvbuf[o] = dinv
                diagbuf[o] = jnp.where(nid < N, dinv * lw, 0.0)
        pltpu.sync_copy(dinvbuf, sh_dinv.at[pl.ds(node0, 640)])

        @pl.when(cid == 0)
        def _():
            pltpu.sync_copy(diagbuf, as_hbm.at[pl.ds(E + node0, 640)])

        @pl.when(cid == 1)
        def _():
            pltpu.sync_copy(diagbuf, at_hbm.at[pl.ds(E + node0, 640)])

        for p in range(TAIL // 16):
            diagbuf[pl.ds(p * 16, 16)] = z16
        tail0 = E + NP + sid * TAIL

        @pl.when(cid == 0)
        def _():
            pltpu.sync_copy(diagbuf.at[pl.ds(0, TAIL)],
                            as_hbm.at[pl.ds(tail0, TAIL)])

        @pl.when(cid == 1)
        def _():
            pltpu.sync_copy(diagbuf.at[pl.ds(0, TAIL)],
                            at_hbm.at[pl.ds(tail0, TAIL)])

        plsc.subcore_barrier()
        pltpu.sync_copy(sh_dinv, loc_dinv)

        @pl.loop(0, ET0 // CH)
        def _(ch):
            off = base_e + ch * CH
            pltpu.sync_copy(r_hbm.at[pl.ds(off, CH)], rbuf)
            pltpu.sync_copy(c_hbm.at[pl.ds(off, CH)], cbuf)
            pltpu.sync_copy(w_hbm.at[pl.ds(off, CH)], wbuf)

            @pl.loop(0, CH // 16)
            def _(g):
                sl = pl.ds(g * 16, 16)
                r = rbuf[sl]
                c = cbuf[sl]
                wv = wbuf[sl]
                ew = jnp.where(r == c, 0.0, wv)
                dst = jnp.where(cid == 0, r, c)
                abuf[sl] = plsc.load_gather(loc_dinv, [dst]) * ew

            @pl.when(cid == 0)
            def _():
                pltpu.sync_copy(abuf, as_hbm.at[pl.ds(off, CH)])

            @pl.when(cid == 1)
            def _():
                pltpu.sync_copy(abuf, at_hbm.at[pl.ds(off, CH)])

    return k(ei[0], ei[1], w)


def _conv(xL, xR, src3, dst3, a3, zeros):
    @functools.partial(
        pl.kernel,
        out_type=jax.ShapeDtypeStruct((2, 2, NP, H), jnp.float32),
        mesh=_MESH,
        compiler_params=_CP,
        scratch_types=[
            pltpu.VMEM((CPT, K), jnp.int32),
            pltpu.VMEM((CPT, K), jnp.int32),
            pltpu.VMEM((CPT, K), jnp.float32),
            pltpu.VMEM((K, H), jnp.float32),
            pltpu.VMEM_SHARED((NP, H), jnp.float32),
            pltpu.SemaphoreType.DMA,
        ],
    )
    def k(xL_hbm, xR_hbm, src_hbm, dst_hbm, a_hbm, z_hbm, part_hbm,
          csrc, cdst, cav, rows, acc, sem):
        cid = lax.axis_index("c")
        sid = lax.axis_index("s")
        wid = cid * 16 + sid
        r0 = sid * 640
        pltpu.sync_copy(src_hbm.at[wid], csrc)
        pltpu.sync_copy(dst_hbm.at[wid], cdst)
        pltpu.sync_copy(a_hbm.at[wid], cav)
        for f, x_hbm in ((0, xL_hbm), (1, xR_hbm)):
            pltpu.sync_copy(z_hbm.at[pl.ds(r0, 640)], acc.at[pl.ds(r0, 640)])
            plsc.subcore_barrier()

            @pl.loop(0, CPT)
            def _(ci):
                pltpu.async_copy(x_hbm.at[csrc.at[ci]], rows, sem).wait()
                ci16 = lax.broadcast(ci, (16,))
                for j in range(K):
                    s = plsc.load_gather(
                        cav, [ci16, jnp.full((16,), j, jnp.int32)])
                    for g in range(4):
                        sl = pl.ds(g * 16, 16)
                        rows[j, sl] = rows[j, sl] * s
                pltpu.sync_copy(rows, acc.at[cdst.at[ci]], add=True)

            plsc.subcore_barrier()
            pltpu.sync_copy(acc.at[pl.ds(r0, 640)],
                            part_hbm.at[cid, f, pl.ds(r0, 640)])

    return k(xL, xR, src3, dst3, a3, zeros)


def _combine(p):
    def body(p_ref, oL_ref, oR_ref):
        oL_ref[...] = p_ref[0, 0] + p_ref[1, 0]
        oR_ref[...] = p_ref[0, 1] + p_ref[1, 1]

    blk = pl.BlockSpec((1024, H), lambda i: (i, 0))
    return pl.pallas_call(
        body,
        grid=(10,),
        in_specs=[pl.BlockSpec((2, 2, 1024, H), lambda i: (0, 0, i, 0))],
        out_specs=[blk, blk],
        out_shape=[jax.ShapeDtypeStruct((NP, H), jnp.float32)] * 2,
    )(p)


def _final(wall, xsL, xsR, y1sL, y1sR, p2s, xtL, xtR, y1tL, y1tR, p2t):
    def body(w_ref, xsL_ref, xsR_ref, y1sL_ref, y1sR_ref, p2s_ref,
             xtL_ref, xtR_ref, y1tL_ref, y1tR_ref, p2t_ref,
             fsL_ref, fsR_ref, ftL_ref, ftR_ref):
        fsL_ref[...] = (w_ref[0] * xsL_ref[...] + w_ref[1] * y1sL_ref[...]
                        + w_ref[2] * (p2s_ref[0, 0] + p2s_ref[1, 0]))
        fsR_ref[...] = (w_ref[0] * xsR_ref[...] + w_ref[1] * y1sR_ref[...]
                        + w_ref[2] * (p2s_ref[0, 1] + p2s_ref[1, 1]))
        ftL_ref[...] = (w_ref[3] * xtL_ref[...] + w_ref[4] * y1tL_ref[...]
                        + w_ref[5] * (p2t_ref[0, 0] + p2t_ref[1, 0]))
        ftR_ref[...] = (w_ref[3] * xtR_ref[...] + w_ref[4] * y1tR_ref[...]
                        + w_ref[5] * (p2t_ref[0, 1] + p2t_ref[1, 1]))

    blk = pl.BlockSpec((1024, H), lambda i: (i, 0))
    blk4 = pl.BlockSpec((2, 2, 1024, H), lambda i: (0, 0, i, 0))
    return pl.pallas_call(
        body,
        grid=(10,),
        in_specs=[pl.BlockSpec(memory_space=pltpu.SMEM),
                  blk, blk, blk, blk, blk4, blk, blk, blk, blk, blk4],
        out_specs=[blk, blk, blk, blk],
        out_shape=[jax.ShapeDtypeStruct((NP, H), jnp.float32)] * 4,
    )(wall, xsL, xsR, y1sL, y1sR, p2s, xtL, xtR, y1tL, y1tR, p2t)


def kernel(x_s, x_t, edge_index, edge_weight, w_s, w_t):
    ei = edge_index.astype(jnp.int32)
    w = edge_weight.astype(jnp.float32)
    a_s, a_t = _phase0(ei, w)

    row, col = ei[0], ei[1]
    ar = jnp.arange(N, dtype=jnp.int32)
    zp = (jnp.arange(EP - E - N, dtype=jnp.int32) * 97) % N
    src_s = jnp.concatenate([col, ar, zp]).reshape(W, CPT, K)
    dst_s = jnp.concatenate([row, ar, zp]).reshape(W, CPT, K)
    src_t = jnp.concatenate([row, ar, zp]).reshape(W, CPT, K)
    dst_t = jnp.concatenate([col, ar, zp]).reshape(W, CPT, K)
    a_s3 = a_s.reshape(W, CPT, K)
    a_t3 = a_t.reshape(W, CPT, K)

    zeros = jnp.zeros((NP, H), jnp.float32)
    pad = jnp.zeros((NP - N, H), jnp.float32)
    xsL = jnp.concatenate([x_s[:, :H], pad])
    xsR = jnp.concatenate([x_s[:, H:], pad])
    xtL = jnp.concatenate([x_t[:, :H], pad])
    xtR = jnp.concatenate([x_t[:, H:], pad])

    p1s = _conv(xsL, xsR, src_s, dst_s, a_s3, zeros)
    p1t = _conv(xtL, xtR, src_t, dst_t, a_t3, zeros)
    y1sL, y1sR = _combine(p1s)
    y1tL, y1tR = _combine(p1t)
    p2s = _conv(y1sL, y1sR, src_s, dst_s, a_s3, zeros)
    p2t = _conv(y1tL, y1tR, src_t, dst_t, a_t3, zeros)

    wall = jnp.concatenate([w_s.astype(jnp.float32)[:, 0],
                            w_t.astype(jnp.float32)[:, 0]])
    fsL, fsR, ftL, ftR = _final(wall, xsL, xsR, y1sL, y1sR, p2s,
                                xtL, xtR, y1tL, y1tR, p2t)
    return jnp.concatenate([fsL[:N], fsR[:N], ftL[:N], ftR[:N]], axis=1)


# speedup vs baseline: 1.8514x; 1.0096x over previous
import dataclasses
import functools

import jax
import jax.numpy as jnp
from jax import lax
from jax.experimental import pallas as pl
from jax.experimental.pallas import tpu as pltpu
from jax.experimental.pallas import tpu_sc as plsc

N = 10000
D = 128
H = 64
E = 320000
FILL = 0.5

NP = 10240
W = 32
K = 128
CPT = 81
EP = W * K * CPT
TAIL = (EP - E - NP) // 16
ET0 = E // 16
CH = 2000

_MESH = plsc.VectorSubcoreMesh(core_axis_name="c", subcore_axis_name="s")
_CP = pltpu.CompilerParams()
if "needs_layout_passes" in pltpu.CompilerParams.__dataclass_fields__:
    _CP = dataclasses.replace(_CP, needs_layout_passes=False)
if "use_tc_tiling_on_sc" in pltpu.CompilerParams.__dataclass_fields__:
    _CP = dataclasses.replace(_CP, use_tc_tiling_on_sc=False)


def _phase0(ei, w):
    @functools.partial(
        pl.kernel,
        out_type=(jax.ShapeDtypeStruct((EP,), jnp.float32),
                  jax.ShapeDtypeStruct((EP,), jnp.float32)),
        mesh=_MESH,
        compiler_params=_CP,
        scratch_types=[
            pltpu.VMEM((160, 64), jnp.float32),
            pltpu.VMEM((160, 64), jnp.float32),
            pltpu.VMEM((160, 64), jnp.float32),
            pltpu.VMEM((NP,), jnp.float32),
            pltpu.VMEM((CH,), jnp.int32),
            pltpu.VMEM((CH,), jnp.int32),
            pltpu.VMEM((CH,), jnp.float32),
            pltpu.VMEM((CH,), jnp.float32),
            pltpu.VMEM((640,), jnp.float32),
            pltpu.VMEM((640,), jnp.float32),
            pltpu.VMEM((10, 64), jnp.float32),
            pltpu.VMEM((160,), jnp.int32),
            pltpu.VMEM_SHARED((160, 64), jnp.float32),
            pltpu.VMEM_SHARED((160, 64), jnp.float32),
            pltpu.VMEM_SHARED((160, 64), jnp.float32),
            pltpu.VMEM_SHARED((NP,), jnp.float32),
        ],
    )
    def k(r_hbm, c_hbm, w_hbm, as_hbm, at_hbm, loc_deg, loc_cnt, loc_sw,
          loc_dinv, rbuf, cbuf, wbuf, abuf, diagbuf, dinvbuf, zbuf, idxref,
          sh_deg, sh_cnt, sh_sw, sh_dinv):
        cid = lax.axis_index("c")
        sid = lax.axis_index("s")
        z16 = jnp.zeros((16,), jnp.float32)
        iota = lax.iota(jnp.int32, 16)

        @pl.loop(0, 160)
        def _(i):
            for g in range(4):
                sl = pl.ds(g * 16, 16)
                loc_deg[i, sl] = z16
                loc_cnt[i, sl] = z16
                loc_sw[i, sl] = z16

        for p in range(10):
            for g in range(4):
                zbuf[p, pl.ds(g * 16, 16)] = z16
        for p in range(10):
            idxref[pl.ds(p * 16, 16)] = p * 16 + iota

        pltpu.sync_copy(zbuf, sh_deg.at[pl.ds(sid * 10, 10)])
        pltpu.sync_copy(zbuf, sh_cnt.at[pl.ds(sid * 10, 10)])
        pltpu.sync_copy(zbuf, sh_sw.at[pl.ds(sid * 10, 10)])
        plsc.subcore_barrier()

        base_e = sid * ET0

        @pl.loop(0, ET0 // CH)
        def _(ch):
            off = base_e + ch * CH
            pltpu.sync_copy(r_hbm.at[pl.ds(off, CH)], rbuf)
            pltpu.sync_copy(c_hbm.at[pl.ds(off, CH)], cbuf)
            pltpu.sync_copy(w_hbm.at[pl.ds(off, CH)], wbuf)

            @pl.loop(0, CH // 16)
            def _(g):
                sl = pl.ds(g * 16, 16)
                r = rbuf[sl]
                c = cbuf[sl]
                wv = wbuf[sl]
                m = r == c
                ew = jnp.where(m, 0.0, wv)
                dst = jnp.where(cid == 0, r, c)
                plsc.addupdate_scatter(loc_deg, [dst >> 6, dst & 63], ew)
                plsc.addupdate_scatter(
                    loc_cnt, [r >> 6, r & 63], jnp.where(m, 1.0, 0.0))
                plsc.addupdate_scatter(
                    loc_sw, [r >> 6, r & 63], jnp.where(m, wv, 0.0))

        pltpu.sync_copy(loc_deg, sh_deg.at[idxref], add=True)
        pltpu.sync_copy(loc_cnt, sh_cnt.at[idxref], add=True)
        pltpu.sync_copy(loc_sw, sh_sw.at[idxref], add=True)
        plsc.subcore_barrier()

        node0 = sid * 640
        pltpu.sync_copy(sh_deg.at[pl.ds(sid * 10, 10)],
                        loc_deg.at[pl.ds(0, 10)])
        pltpu.sync_copy(sh_cnt.at[pl.ds(sid * 10, 10)],
                        loc_cnt.at[pl.ds(0, 10)])
        pltpu.sync_copy(sh_sw.at[pl.ds(sid * 10, 10)],
                        loc_sw.at[pl.ds(0, 10)])
        for p in range(10):
            for g in range(4):
                sl = pl.ds(g * 16, 16)
                dg = loc_deg[p, sl]
                cn = loc_cnt[p, sl]
                sw = loc_sw[p, sl]
                has = cn > 0.0
                lw = jnp.where(has, sw / jnp.where(has, cn, 1.0), FILL)
                degt = dg + lw
                dinv = jnp.where(degt == 0.0, 0.0, 1.0 / degt)
                nid = node0 + p * 64 + g * 16 + iota
                o = pl.ds(p * 64 + g * 16, 16)
                dinvbuf[o] = dinv
                diagbuf[o] = jnp.where(nid < N, dinv * lw, 0.0)
        pltpu.sync_copy(dinvbuf, sh_dinv.at[pl.ds(node0, 640)])

        @pl.when(cid == 0)
        def _():
            pltpu.sync_copy(diagbuf, as_hbm.at[pl.ds(E + node0, 640)])

        @pl.when(cid == 1)
        def _():
            pltpu.sync_copy(diagbuf, at_hbm.at[pl.ds(E + node0, 640)])

        for p in range(TAIL // 16):
            diagbuf[pl.ds(p * 16, 16)] = z16
        tail0 = E + NP + sid * TAIL

        @pl.when(cid == 0)
        def _():
            pltpu.sync_copy(diagbuf.at[pl.ds(0, TAIL)],
                            as_hbm.at[pl.ds(tail0, TAIL)])

        @pl.when(cid == 1)
        def _():
            pltpu.sync_copy(diagbuf.at[pl.ds(0, TAIL)],
                            at_hbm.at[pl.ds(tail0, TAIL)])

        plsc.subcore_barrier()
        pltpu.sync_copy(sh_dinv, loc_dinv)

        @pl.loop(0, ET0 // CH)
        def _(ch):
            off = base_e + ch * CH
            pltpu.sync_copy(r_hbm.at[pl.ds(off, CH)], rbuf)
            pltpu.sync_copy(c_hbm.at[pl.ds(off, CH)], cbuf)
            pltpu.sync_copy(w_hbm.at[pl.ds(off, CH)], wbuf)

            @pl.loop(0, CH // 16)
            def _(g):
                sl = pl.ds(g * 16, 16)
                r = rbuf[sl]
                c = cbuf[sl]
                wv = wbuf[sl]
                ew = jnp.where(r == c, 0.0, wv)
                dst = jnp.where(cid == 0, r, c)
                abuf[sl] = plsc.load_gather(loc_dinv, [dst]) * ew

            @pl.when(cid == 0)
            def _():
                pltpu.sync_copy(abuf, as_hbm.at[pl.ds(off, CH)])

            @pl.when(cid == 1)
            def _():
                pltpu.sync_copy(abuf, at_hbm.at[pl.ds(off, CH)])

    return k(ei[0], ei[1], w)


def _conv(xL, xR, src3, dst3, a3, zeros):
    @functools.partial(
        pl.kernel,
        out_type=jax.ShapeDtypeStruct((2, 2, NP, H), jnp.float32),
        mesh=_MESH,
        compiler_params=_CP,
        scratch_types=[
            pltpu.VMEM((CPT, K), jnp.int32),
            pltpu.VMEM((CPT, K), jnp.int32),
            pltpu.VMEM((CPT, K), jnp.float32),
            pltpu.VMEM((K, H), jnp.float32),
            pltpu.VMEM_SHARED((NP, H), jnp.float32),
            pltpu.SemaphoreType.DMA,
        ],
    )
    def k(xL_hbm, xR_hbm, src_hbm, dst_hbm, a_hbm, z_hbm, part_hbm,
          csrc, cdst, cav, rows, acc, sem):
        cid = lax.axis_index("c")
        sid = lax.axis_index("s")
        wid = cid * 16 + sid
        r0 = sid * 640
        pltpu.sync_copy(src_hbm.at[wid], csrc)
        pltpu.sync_copy(dst_hbm.at[wid], cdst)
        pltpu.sync_copy(a_hbm.at[wid], cav)
        for f, x_hbm in ((0, xL_hbm), (1, xR_hbm)):
            pltpu.sync_copy(z_hbm.at[pl.ds(r0, 640)], acc.at[pl.ds(r0, 640)])
            plsc.subcore_barrier()

            @pl.loop(0, CPT)
            def _(ci):
                pltpu.async_copy(x_hbm.at[csrc.at[ci]], rows, sem).wait()
                ci16 = lax.broadcast(ci, (16,))
                for j in range(K):
                    s = plsc.load_gather(
                        cav, [ci16, jnp.full((16,), j, jnp.int32)])
                    for g in range(4):
                        sl = pl.ds(g * 16, 16)
                        rows[j, sl] = rows[j, sl] * s
                pltpu.sync_copy(rows, acc.at[cdst.at[ci]], add=True)

            plsc.subcore_barrier()
            pltpu.sync_copy(acc.at[pl.ds(r0, 640)],
                            part_hbm.at[cid, f, pl.ds(r0, 640)])

    return k(xL, xR, src3, dst3, a3, zeros)


def _combine(p):
    def body(p_ref, oL_ref, oR_ref):
        oL_ref[...] = p_ref[0, 0] + p_ref[1, 0]
        oR_ref[...] = p_ref[0, 1] + p_ref[1, 1]

    blk = pl.BlockSpec((1024, H), lambda i: (i, 0))
    return pl.pallas_call(
        body,
        grid=(10,),
        in_specs=[pl.BlockSpec((2, 2, 1024, H), lambda i: (0, 0, i, 0))],
        out_specs=[blk, blk],
        out_shape=[jax.ShapeDtypeStruct((NP, H), jnp.float32)] * 2,
    )(p)


def _final(wall, xsL, xsR, y1sL, y1sR, p2s, xtL, xtR, y1tL, y1tR, p2t):
    def body(w_ref, xsL_ref, xsR_ref, y1sL_ref, y1sR_ref, p2s_ref,
             xtL_ref, xtR_ref, y1tL_ref, y1tR_ref, p2t_ref,
             fsL_ref, fsR_ref, ftL_ref, ftR_ref):
        fsL_ref[...] = (w_ref[0] * xsL_ref[...] + w_ref[1] * y1sL_ref[...]
                        + w_ref[2] * (p2s_ref[0, 0] + p2s_ref[1, 0]))
        fsR_ref[...] = (w_ref[0] * xsR_ref[...] + w_ref[1] * y1sR_ref[...]
                        + w_ref[2] * (p2s_ref[0, 1] + p2s_ref[1, 1]))
        ftL_ref[...] = (w_ref[3] * xtL_ref[...] + w_ref[4] * y1tL_ref[...]
                        + w_ref[5] * (p2t_ref[0, 0] + p2t_ref[1, 0]))
        ftR_ref[...] = (w_ref[3] * xtR_ref[...] + w_ref[4] * y1tR_ref[...]
                        + w_ref[5] * (p2t_ref[0, 1] + p2t_ref[1, 1]))

    blk = pl.BlockSpec((1024, H), lambda i: (i, 0))
    blk4 = pl.BlockSpec((2, 2, 1024, H), lambda i: (0, 0, i, 0))
    return pl.pallas_call(
        body,
        grid=(10,),
        in_specs=[pl.BlockSpec(memory_space=pltpu.SMEM),
                  blk, blk, blk, blk, blk4, blk, blk, blk, blk, blk4],
        out_specs=[blk, blk, blk, blk],
        out_shape=[jax.ShapeDtypeStruct((NP, H), jnp.float32)] * 4,
    )(wall, xsL, xsR, y1sL, y1sR, p2s, xtL, xtR, y1tL, y1tR, p2t)


def kernel(x_s, x_t, edge_index, edge_weight, w_s, w_t):
    ei = edge_index.astype(jnp.int32)
    w = edge_weight.astype(jnp.float32)
    a_s, a_t = _phase0(ei, w)

    row, col = ei[0], ei[1]
    ar = jnp.arange(N, dtype=jnp.int32)
    zp = (jnp.arange(EP - E - N, dtype=jnp.int32) * 97) % N
    src_s = jnp.concatenate([col, ar, zp]).reshape(W, CPT, K)
    dst_s = jnp.concatenate([row, ar, zp]).reshape(W, CPT, K)
    src_t = jnp.concatenate([row, ar, zp]).reshape(W, CPT, K)
    dst_t = jnp.concatenate([col, ar, zp]).reshape(W, CPT, K)
    a_s3 = a_s.reshape(W, CPT, K)
    a_t3 = a_t.reshape(W, CPT, K)

    zeros = jnp.zeros((NP, H), jnp.float32)
    pad = jnp.zeros((NP - N, H), jnp.float32)
    xsL = jnp.concatenate([x_s[:, :H], pad])
    xsR = jnp.concatenate([x_s[:, H:], pad])
    xtL = jnp.concatenate([x_t[:, :H], pad])
    xtR = jnp.concatenate([x_t[:, H:], pad])

    p1s = _conv(xsL, xsR, src_s, dst_s, a_s3, zeros)
    p1t = _conv(xtL, xtR, src_t, dst_t, a_t3, zeros)
    y1sL, y1sR = _combine(p1s)
    y1tL, y1tR = _combine(p1t)
    p2s = _conv(y1sL, y1sR, src_s, dst_s, a_s3, zeros)
    p2t = _conv(y1tL, y1tR, src_t, dst_t, a_t3, zeros)

    wall = jnp.concatenate([w_s.astype(jnp.float32)[:, 0],
                            w_t.astype(jnp.float32)[:, 0]])
    fsL, fsR, ftL, ftR = _final(wall, xsL, xsR, y1sL, y1sR, p2s,
                                xtL, xtR, y1tL, y1tR, p2t)
    return jnp.concatenate([fsL[:N], fsR[:N], ftL[:N], ftR[:N]], axis=1)
